# baseline (speedup 1.0000x reference)
import functools

import jax
import jax.numpy as jnp
from jax import lax
from jax.experimental import pallas as pl
from jax.experimental.pallas import tpu as pltpu
from jax.experimental.pallas import tpu_sc as plsc

_BATCH = 16384
_SEQ = 20
_DIM = 64
_NW = 32
_BG = 128
_GROUPS_PER_W = (_BATCH // _BG) // _NW
_B_PER_W = _BG * _GROUPS_PER_W
_NBLK = _GROUPS_PER_W * _SEQ


def _gather_body(idx_hbm, table_hbm, out_hbm, idx_v, rows_v, trans_v,
                 sem_g, sem_w):
    wid = lax.axis_index("s") * 2 + lax.axis_index("c")

    pltpu.sync_copy(idx_hbm.at[:, pl.ds(wid * _B_PER_W, _B_PER_W)], idx_v)

    iota = lax.iota(jnp.int32, 16)
    b_vecs = [iota + 16 * k for k in range(8)]

    def start_gather(j, buf):
        s, cl = j >> 2, j & 3
        offs = idx_v.at[s, pl.ds(cl * _BG, _BG)]
        return pltpu.async_copy(table_hbm.at[offs], rows_v.at[buf], sem_g)

    def wait_gather(buf):
        pltpu.make_async_copy(
            table_hbm.at[pl.ds(0, _BG)], rows_v.at[buf], sem_g).wait()

    def start_wb(j, buf):
        s, cl = j >> 2, j & 3
        c = wid * _GROUPS_PER_W + cl
        return pltpu.async_copy(
            trans_v.at[buf],
            out_hbm.at[pl.ds(s, 1), pl.ds(0, 8), pl.ds(c, 1)], sem_w)

    def wait_wb(buf):
        pltpu.make_async_copy(
            trans_v.at[buf],
            out_hbm.at[pl.ds(0, 1), pl.ds(0, 8), pl.ds(0, 1)], sem_w).wait()

    zero16 = jnp.full((16,), 0, jnp.int32)

    def transpose(j, buf):
        src = rows_v.at[buf]

        buf_splat = zero16 + buf

        def body(q, carry):
            dlo = (iota + q) & 15
            for m in range(4):
                d_vec = dlo + (16 * m)
                r_vec = d_vec >> 3
                din_vec = d_vec & 7
                for kb in range(8):
                    vals = plsc.load_gather(src, [b_vecs[kb], d_vec])
                    plsc.store_scatter(
                        trans_v,
                        [buf_splat, zero16, r_vec, zero16,
                         din_vec, b_vecs[kb]], vals)
            return carry

        lax.fori_loop(0, 16, body, 0)

    start_gather(0, 0)
    start_gather(1, 1)

    def pair_body(t, carry):
        for buf in (0, 1):
            j = 2 * t + buf
            wait_gather(buf)

            @pl.when(t > 0)
            def _():
                wait_wb(buf)

            transpose(j, buf)

            @pl.when(t < _NBLK // 2 - 1)
            def _():
                start_gather(j + 2, buf)

            start_wb(j, buf)
        return carry

    lax.fori_loop(0, _NBLK // 2, pair_body, 0)
    wait_wb(0)
    wait_wb(1)


_gather = functools.partial(
    pl.kernel,
    mesh=plsc.VectorSubcoreMesh(core_axis_name="c", subcore_axis_name="s"),
    compiler_params=pltpu.CompilerParams(use_tc_tiling_on_sc=True,
                                         needs_layout_passes=False),
    out_type=jax.ShapeDtypeStruct((_SEQ, 8, _BATCH // _BG, 8, _BG),
                                  jnp.float32),
    scratch_types=[
        pltpu.VMEM((_SEQ, _B_PER_W), jnp.int32),
        pltpu.VMEM((2, _BG, 128), jnp.float32),
        pltpu.VMEM((2, 1, 8, 1, 8, _BG), jnp.float32),
        pltpu.SemaphoreType.DMA,
        pltpu.SemaphoreType.DMA,
    ],
)(_gather_body)


@jax.jit
def kernel(indexes, entity_table):
    out5 = _gather(indexes.T, jnp.pad(entity_table, ((0, 0), (0, 64))))
    return out5.transpose(2, 4, 0, 1, 3).reshape(_BATCH, _SEQ, _DIM)
